# Initial kernel scaffold; baseline (speedup 1.0000x reference)
import jax
import jax.numpy as jnp
from jax.experimental import pallas as pl


def kernel(h, edge_index, coord, edge_attr, node_ln_g, node_ln_b, edge_ln_g, edge_ln_b, e_W1, e_b1, e_W2, e_b2, n_W1, n_b1, n_W2, n_b2, c_W1, c_b1, c_W2):
    raise NotImplementedError("write your pallas kernel here")


# speedup vs baseline: 1.2519x; 1.2519x over previous
import functools

import jax
import jax.numpy as jnp
from jax.experimental import pallas as pl
from jax.experimental.pallas import tpu as pltpu

N = 10000
E = 320000
D = 128
H = 128
DE = 16

BN = 2000
BE = 2000


def _silu(x):
    return x / (1.0 + jnp.exp(-x))


def _ln(x, g, b, eps=1e-5):
    mu = jnp.mean(x, axis=-1, keepdims=True)
    var = jnp.mean((x - mu) ** 2, axis=-1, keepdims=True)
    return (x - mu) * jax.lax.rsqrt(var + eps) * g + b


def _k1_body(h_ref, coord_ref, g_ref, b_ref, w1r_ref, w1c_ref,
             hn_ref, pr_ref, pc_ref, c8_ref):
    h = h_ref[:]
    hn = _ln(h, g_ref[:], b_ref[:])
    hn_ref[:] = hn
    pr_ref[:] = jnp.dot(hn, w1r_ref[:], preferred_element_type=jnp.float32)
    pc_ref[:] = jnp.dot(hn, w1c_ref[:], preferred_element_type=jnp.float32)
    c = coord_ref[:]
    c8_ref[:] = jnp.pad(c, ((0, 0), (0, 5)))


def _k1(h, coord, g, b, w1r, w1c):
    grid = N // BN
    return pl.pallas_call(
        _k1_body,
        grid=(grid,),
        in_specs=[
            pl.BlockSpec((BN, D), lambda i: (i, 0)),
            pl.BlockSpec((BN, 3), lambda i: (i, 0)),
            pl.BlockSpec((D,), lambda i: (0,)),
            pl.BlockSpec((D,), lambda i: (0,)),
            pl.BlockSpec((D, H), lambda i: (0, 0)),
            pl.BlockSpec((D, H), lambda i: (0, 0)),
        ],
        out_specs=[
            pl.BlockSpec((BN, D), lambda i: (i, 0)),
            pl.BlockSpec((BN, H), lambda i: (i, 0)),
            pl.BlockSpec((BN, H), lambda i: (i, 0)),
            pl.BlockSpec((BN, 8), lambda i: (i, 0)),
        ],
        out_shape=[
            jax.ShapeDtypeStruct((N, D), jnp.float32),
            jax.ShapeDtypeStruct((N, H), jnp.float32),
            jax.ShapeDtypeStruct((N, H), jnp.float32),
            jax.ShapeDtypeStruct((N, 8), jnp.float32),
        ],
    )(h, coord, g, b, w1r, w1c)


def _k3_body(gr_ref, gc_ref, cr_ref, cc_ref, ea_ref,
             wrad_ref, w1e_ref, b1_ref, w2_ref, b2_ref,
             lng_ref, lnb_ref, cw1_ref, cb1_ref, cw2_ref,
             ef_ref, tr_ref):
    cdiff = cr_ref[:] - cc_ref[:]
    radial = jnp.sum(cdiff * cdiff, axis=1, keepdims=True)
    pre1 = (gr_ref[:] + gc_ref[:]
            + radial * wrad_ref[:]
            + jnp.dot(ea_ref[:], w1e_ref[:], preferred_element_type=jnp.float32)
            + b1_ref[:])
    x1 = _silu(pre1)
    x2 = _silu(jnp.dot(x1, w2_ref[:], preferred_element_type=jnp.float32) + b2_ref[:])
    ef = _ln(x2, lng_ref[:], lnb_ref[:])
    ef_ref[:] = ef
    cm = jnp.dot(_silu(jnp.dot(ef, cw1_ref[:], preferred_element_type=jnp.float32) + cb1_ref[:]),
                 cw2_ref[:], preferred_element_type=jnp.float32)
    tr = cdiff * cm
    col = jax.lax.broadcasted_iota(jnp.int32, (BE, 8), 1)
    tr_ref[:] = jnp.where(col == 3, 1.0, tr)


def _k3(gr, gc, cr, cc, ea, wrad, w1e, b1, w2, b2, lng, lnb, cw1, cb1, cw2):
    grid = E // BE
    full = lambda shape: pl.BlockSpec(shape, lambda i: tuple(0 for _ in shape))
    return pl.pallas_call(
        _k3_body,
        grid=(grid,),
        in_specs=[
            pl.BlockSpec((BE, H), lambda i: (i, 0)),
            pl.BlockSpec((BE, H), lambda i: (i, 0)),
            pl.BlockSpec((BE, 8), lambda i: (i, 0)),
            pl.BlockSpec((BE, 8), lambda i: (i, 0)),
            pl.BlockSpec((BE, DE), lambda i: (i, 0)),
            full((1, H)),
            full((DE, H)),
            full((H,)),
            full((H, H)),
            full((H,)),
            full((H,)),
            full((H,)),
            full((H, H)),
            full((H,)),
            full((H, 1)),
        ],
        out_specs=[
            pl.BlockSpec((BE, H), lambda i: (i, 0)),
            pl.BlockSpec((BE, 8), lambda i: (i, 0)),
        ],
        out_shape=[
            jax.ShapeDtypeStruct((E, H), jnp.float32),
            jax.ShapeDtypeStruct((E, 8), jnp.float32),
        ],
    )(gr, gc, cr, cc, ea, wrad, w1e, b1, w2, b2, lng, lnb, cw1, cb1, cw2)


def _k5_body(h_ref, hn_ref, agg_ref, tr_ref, coord_ref,
             w1h_ref, w1a_ref, b1_ref, w2_ref, b2_ref,
             hout_ref, cout_ref):
    agg = agg_ref[0] + agg_ref[1]
    tr = tr_ref[0] + tr_ref[1]
    pre = (jnp.dot(hn_ref[:], w1h_ref[:], preferred_element_type=jnp.float32)
           + jnp.dot(agg, w1a_ref[:], preferred_element_type=jnp.float32)
           + b1_ref[:])
    nh = jnp.dot(_silu(pre), w2_ref[:], preferred_element_type=jnp.float32) + b2_ref[:]
    hout_ref[:] = h_ref[:] + nh
    cnt = jnp.maximum(tr[:, 3:4], 1.0)
    cout_ref[:] = coord_ref[:] + tr[:, 0:3] / cnt


def _k5(h, hn, aggp, trp, coord, w1h, w1a, b1, w2, b2):
    grid = N // BN
    full = lambda shape: pl.BlockSpec(shape, lambda i: tuple(0 for _ in shape))
    return pl.pallas_call(
        _k5_body,
        grid=(grid,),
        in_specs=[
            pl.BlockSpec((BN, D), lambda i: (i, 0)),
            pl.BlockSpec((BN, D), lambda i: (i, 0)),
            pl.BlockSpec((2, BN, H), lambda i: (0, i, 0)),
            pl.BlockSpec((2, BN, 8), lambda i: (0, i, 0)),
            pl.BlockSpec((BN, 3), lambda i: (i, 0)),
            full((D, H)),
            full((H, H)),
            full((H,)),
            full((H, D)),
            full((D,)),
        ],
        out_specs=[
            pl.BlockSpec((BN, D), lambda i: (i, 0)),
            pl.BlockSpec((BN, 3), lambda i: (i, 0)),
        ],
        out_shape=[
            jax.ShapeDtypeStruct((N, D), jnp.float32),
            jax.ShapeDtypeStruct((N, 3), jnp.float32),
        ],
    )(h, hn, aggp, trp, coord, w1h, w1a, b1, w2, b2)


def kernel(h, edge_index, coord, edge_attr, node_ln_g, node_ln_b,
           edge_ln_g, edge_ln_b, e_W1, e_b1, e_W2, e_b2,
           n_W1, n_b1, n_W2, n_b2, c_W1, c_b1, c_W2):
    row = edge_index[0]
    col = edge_index[1]
    w1r = e_W1[0:D]
    w1c = e_W1[D:2 * D]
    wrad = e_W1[2 * D:2 * D + 1]
    w1e = e_W1[2 * D + 1:]

    hn, pr, pc, c8 = _k1(h, coord, node_ln_g, node_ln_b, w1r, w1c)

    gr = pr[row]
    gc = pc[col]
    cr = c8[row]
    cc = c8[col]

    ef, tr8 = _k3(gr, gc, cr, cc, edge_attr, wrad, w1e, e_b1, e_W2, e_b2,
                  edge_ln_g, edge_ln_b, c_W1, c_b1, c_W2)

    agg = jax.ops.segment_sum(ef, row, num_segments=N)
    trs = jax.ops.segment_sum(tr8, row, num_segments=N)
    aggp = jnp.stack([agg, jnp.zeros_like(agg)])
    trp = jnp.stack([trs, jnp.zeros_like(trs)])

    h_out, coord_out = _k5(h, hn, aggp, trp, coord,
                           n_W1[0:D], n_W1[D:], n_b1, n_W2, n_b2)
    return (h_out, coord_out, edge_attr)


# speedup vs baseline: 2.2699x; 1.8131x over previous
import functools

import jax
import jax.numpy as jnp
from jax import lax
from jax.experimental import pallas as pl
from jax.experimental.pallas import tpu as pltpu
from jax.experimental.pallas import tpu_sc as plsc

N = 10000
E = 320000
D = 128
H = 128
DE = 16

BN = 2000
BE = 2000

_NC = 2
_NS = 16
_NW = _NC * _NS
_CG = 80
_EPW = E // _NW
_NCH = _EPW // _CG


def _silu(x):
    return x / (1.0 + jnp.exp(-x))


def _ln(x, g, b, eps=1e-5):
    mu = jnp.mean(x, axis=-1, keepdims=True)
    var = jnp.mean((x - mu) ** 2, axis=-1, keepdims=True)
    return (x - mu) * jax.lax.rsqrt(var + eps) * g + b


def _k1_body(h_ref, coord_ref, g_ref, b_ref, w1r_ref, w1c_ref,
             hn_ref, pr_ref, pc_ref, c8_ref):
    h = h_ref[:]
    hn = _ln(h, g_ref[:], b_ref[:])
    hn_ref[:] = hn
    pr_ref[:] = jnp.dot(hn, w1r_ref[:], preferred_element_type=jnp.float32)
    pc_ref[:] = jnp.dot(hn, w1c_ref[:], preferred_element_type=jnp.float32)
    c = coord_ref[:]
    c8_ref[:] = jnp.pad(c, ((0, 0), (0, 5)))


def _k1(h, coord, g, b, w1r, w1c):
    grid = N // BN
    return pl.pallas_call(
        _k1_body,
        grid=(grid,),
        in_specs=[
            pl.BlockSpec((BN, D), lambda i: (i, 0)),
            pl.BlockSpec((BN, 3), lambda i: (i, 0)),
            pl.BlockSpec((D,), lambda i: (0,)),
            pl.BlockSpec((D,), lambda i: (0,)),
            pl.BlockSpec((D, H), lambda i: (0, 0)),
            pl.BlockSpec((D, H), lambda i: (0, 0)),
        ],
        out_specs=[
            pl.BlockSpec((BN, D), lambda i: (i, 0)),
            pl.BlockSpec((BN, H), lambda i: (i, 0)),
            pl.BlockSpec((BN, H), lambda i: (i, 0)),
            pl.BlockSpec((BN, 8), lambda i: (i, 0)),
        ],
        out_shape=[
            jax.ShapeDtypeStruct((N, D), jnp.float32),
            jax.ShapeDtypeStruct((N, H), jnp.float32),
            jax.ShapeDtypeStruct((N, H), jnp.float32),
            jax.ShapeDtypeStruct((N, 8), jnp.float32),
        ],
    )(h, coord, g, b, w1r, w1c)


def _k2_body(pr, pc, c8, row, col, gr, gc, cr, cc,
             rowb, colb, grb, gcb, crb, ccb, sem):
    wid = lax.axis_index("s") * _NC + lax.axis_index("c")
    base = wid * _EPW
    pltpu.sync_copy(row.at[pl.ds(base, _EPW)], rowb)
    pltpu.sync_copy(col.at[pl.ds(base, _EPW)], colb)

    def chunk(i, carry):
        off = i * _CG
        idx_r = rowb.at[pl.ds(off, _CG)]
        idx_c = colb.at[pl.ds(off, _CG)]
        c1 = pltpu.async_copy(pr.at[idx_r], grb, sem)
        c2 = pltpu.async_copy(pc.at[idx_c], gcb, sem)
        c3 = pltpu.async_copy(c8.at[idx_r], crb, sem)
        c4 = pltpu.async_copy(c8.at[idx_c], ccb, sem)
        c1.wait()
        c2.wait()
        c3.wait()
        c4.wait()
        pltpu.sync_copy(grb, gr.at[pl.ds(base + off, _CG)])
        pltpu.sync_copy(gcb, gc.at[pl.ds(base + off, _CG)])
        pltpu.sync_copy(crb, cr.at[pl.ds(base + off, _CG)])
        pltpu.sync_copy(ccb, cc.at[pl.ds(base + off, _CG)])
        return carry

    lax.fori_loop(0, _NCH, chunk, 0)


def _k2(pr, pc, c8, row, col):
    mesh = plsc.VectorSubcoreMesh(core_axis_name="c", subcore_axis_name="s")
    f = pl.kernel(
        _k2_body,
        out_type=[
            jax.ShapeDtypeStruct((E, H), jnp.float32),
            jax.ShapeDtypeStruct((E, H), jnp.float32),
            jax.ShapeDtypeStruct((E, 8), jnp.float32),
            jax.ShapeDtypeStruct((E, 8), jnp.float32),
        ],
        mesh=mesh,
        scratch_types=[
            pltpu.VMEM((_EPW,), jnp.int32),
            pltpu.VMEM((_EPW,), jnp.int32),
            pltpu.VMEM((_CG, H), jnp.float32),
            pltpu.VMEM((_CG, H), jnp.float32),
            pltpu.VMEM((_CG, 8), jnp.float32),
            pltpu.VMEM((_CG, 8), jnp.float32),
            pltpu.SemaphoreType.DMA,
        ],
        compiler_params=pltpu.CompilerParams(use_tc_tiling_on_sc=False),
    )
    return f(pr, pc, c8, row, col)


def _k3_body(gr_ref, gc_ref, cr_ref, cc_ref, ea_ref,
             wrad_ref, w1e_ref, b1_ref, w2_ref, b2_ref,
             lng_ref, lnb_ref, cw1_ref, cb1_ref, cw2_ref,
             ef_ref, tr_ref):
    cdiff = cr_ref[:] - cc_ref[:]
    radial = jnp.sum(cdiff * cdiff, axis=1, keepdims=True)
    pre1 = (gr_ref[:] + gc_ref[:]
            + radial * wrad_ref[:]
            + jnp.dot(ea_ref[:], w1e_ref[:], preferred_element_type=jnp.float32)
            + b1_ref[:])
    x1 = _silu(pre1)
    x2 = _silu(jnp.dot(x1, w2_ref[:], preferred_element_type=jnp.float32) + b2_ref[:])
    ef = _ln(x2, lng_ref[:], lnb_ref[:])
    ef_ref[:] = ef
    cm = jnp.dot(_silu(jnp.dot(ef, cw1_ref[:], preferred_element_type=jnp.float32) + cb1_ref[:]),
                 cw2_ref[:], preferred_element_type=jnp.float32)
    tr = cdiff * cm
    col = jax.lax.broadcasted_iota(jnp.int32, (BE, 8), 1)
    tr_ref[:] = jnp.where(col == 3, 1.0, tr)


def _k3(gr, gc, cr, cc, ea, wrad, w1e, b1, w2, b2, lng, lnb, cw1, cb1, cw2):
    grid = E // BE
    full = lambda shape: pl.BlockSpec(shape, lambda i: tuple(0 for _ in shape))
    return pl.pallas_call(
        _k3_body,
        grid=(grid,),
        in_specs=[
            pl.BlockSpec((BE, H), lambda i: (i, 0)),
            pl.BlockSpec((BE, H), lambda i: (i, 0)),
            pl.BlockSpec((BE, 8), lambda i: (i, 0)),
            pl.BlockSpec((BE, 8), lambda i: (i, 0)),
            pl.BlockSpec((BE, DE), lambda i: (i, 0)),
            full((1, H)),
            full((DE, H)),
            full((H,)),
            full((H, H)),
            full((H,)),
            full((H,)),
            full((H,)),
            full((H, H)),
            full((H,)),
            full((H, 1)),
        ],
        out_specs=[
            pl.BlockSpec((BE, H), lambda i: (i, 0)),
            pl.BlockSpec((BE, 8), lambda i: (i, 0)),
        ],
        out_shape=[
            jax.ShapeDtypeStruct((E, H), jnp.float32),
            jax.ShapeDtypeStruct((E, 8), jnp.float32),
        ],
    )(gr, gc, cr, cc, ea, wrad, w1e, b1, w2, b2, lng, lnb, cw1, cb1, cw2)


def _k5_body(h_ref, hn_ref, agg_ref, tr_ref, coord_ref,
             w1h_ref, w1a_ref, b1_ref, w2_ref, b2_ref,
             hout_ref, cout_ref):
    agg = agg_ref[0] + agg_ref[1]
    tr = tr_ref[0] + tr_ref[1]
    pre = (jnp.dot(hn_ref[:], w1h_ref[:], preferred_element_type=jnp.float32)
           + jnp.dot(agg, w1a_ref[:], preferred_element_type=jnp.float32)
           + b1_ref[:])
    nh = jnp.dot(_silu(pre), w2_ref[:], preferred_element_type=jnp.float32) + b2_ref[:]
    hout_ref[:] = h_ref[:] + nh
    cnt = jnp.maximum(tr[:, 3:4], 1.0)
    cout_ref[:] = coord_ref[:] + tr[:, 0:3] / cnt


def _k5(h, hn, aggp, trp, coord, w1h, w1a, b1, w2, b2):
    grid = N // BN
    full = lambda shape: pl.BlockSpec(shape, lambda i: tuple(0 for _ in shape))
    return pl.pallas_call(
        _k5_body,
        grid=(grid,),
        in_specs=[
            pl.BlockSpec((BN, D), lambda i: (i, 0)),
            pl.BlockSpec((BN, D), lambda i: (i, 0)),
            pl.BlockSpec((2, BN, H), lambda i: (0, i, 0)),
            pl.BlockSpec((2, BN, 8), lambda i: (0, i, 0)),
            pl.BlockSpec((BN, 3), lambda i: (i, 0)),
            full((D, H)),
            full((H, H)),
            full((H,)),
            full((H, D)),
            full((D,)),
        ],
        out_specs=[
            pl.BlockSpec((BN, D), lambda i: (i, 0)),
            pl.BlockSpec((BN, 3), lambda i: (i, 0)),
        ],
        out_shape=[
            jax.ShapeDtypeStruct((N, D), jnp.float32),
            jax.ShapeDtypeStruct((N, 3), jnp.float32),
        ],
    )(h, hn, aggp, trp, coord, w1h, w1a, b1, w2, b2)


def kernel(h, edge_index, coord, edge_attr, node_ln_g, node_ln_b,
           edge_ln_g, edge_ln_b, e_W1, e_b1, e_W2, e_b2,
           n_W1, n_b1, n_W2, n_b2, c_W1, c_b1, c_W2):
    row = edge_index[0]
    col = edge_index[1]
    w1r = e_W1[0:D]
    w1c = e_W1[D:2 * D]
    wrad = e_W1[2 * D:2 * D + 1]
    w1e = e_W1[2 * D + 1:]

    hn, pr, pc, c8 = _k1(h, coord, node_ln_g, node_ln_b, w1r, w1c)

    gr, gc, cr, cc = _k2(pr, pc, c8, row, col)

    ef, tr8 = _k3(gr, gc, cr, cc, edge_attr, wrad, w1e, e_b1, e_W2, e_b2,
                  edge_ln_g, edge_ln_b, c_W1, c_b1, c_W2)

    agg = jax.ops.segment_sum(ef, row, num_segments=N)
    trs = jax.ops.segment_sum(tr8, row, num_segments=N)
    aggp = jnp.stack([agg, jnp.zeros_like(agg)])
    trp = jnp.stack([trs, jnp.zeros_like(trs)])

    h_out, coord_out = _k5(h, hn, aggp, trp, coord,
                           n_W1[0:D], n_W1[D:], n_b1, n_W2, n_b2)
    return (h_out, coord_out, edge_attr)


# speedup vs baseline: 3.7526x; 1.6532x over previous
import functools

import jax
import jax.numpy as jnp
from jax import lax
from jax.experimental import pallas as pl
from jax.experimental.pallas import tpu as pltpu
from jax.experimental.pallas import tpu_sc as plsc

N = 10000
E = 320000
D = 128
H = 128
DE = 16

BN = 2000
BE = 2000

_NC = 2
_NS = 16
_NW = _NC * _NS
_CG = 80
_EPW = E // _NW
_NCH = _EPW // _CG


def _silu(x):
    return x / (1.0 + jnp.exp(-x))


def _ln(x, g, b, eps=1e-5):
    mu = jnp.mean(x, axis=-1, keepdims=True)
    var = jnp.mean((x - mu) ** 2, axis=-1, keepdims=True)
    return (x - mu) * jax.lax.rsqrt(var + eps) * g + b


def _k1_body(h_ref, coord_ref, g_ref, b_ref, w1r_ref, w1c_ref,
             hn_ref, pr_ref, pc_ref, c8_ref):
    h = h_ref[:]
    hn = _ln(h, g_ref[:], b_ref[:])
    hn_ref[:] = hn
    pr_ref[:] = jnp.dot(hn, w1r_ref[:], preferred_element_type=jnp.float32)
    pc_ref[:] = jnp.dot(hn, w1c_ref[:], preferred_element_type=jnp.float32)
    c = coord_ref[:]
    c8_ref[:] = jnp.pad(c, ((0, 0), (0, 5)))


def _k1(h, coord, g, b, w1r, w1c):
    grid = N // BN
    return pl.pallas_call(
        _k1_body,
        grid=(grid,),
        in_specs=[
            pl.BlockSpec((BN, D), lambda i: (i, 0)),
            pl.BlockSpec((BN, 3), lambda i: (i, 0)),
            pl.BlockSpec((D,), lambda i: (0,)),
            pl.BlockSpec((D,), lambda i: (0,)),
            pl.BlockSpec((D, H), lambda i: (0, 0)),
            pl.BlockSpec((D, H), lambda i: (0, 0)),
        ],
        out_specs=[
            pl.BlockSpec((BN, D), lambda i: (i, 0)),
            pl.BlockSpec((BN, H), lambda i: (i, 0)),
            pl.BlockSpec((BN, H), lambda i: (i, 0)),
            pl.BlockSpec((BN, 8), lambda i: (i, 0)),
        ],
        out_shape=[
            jax.ShapeDtypeStruct((N, D), jnp.float32),
            jax.ShapeDtypeStruct((N, H), jnp.float32),
            jax.ShapeDtypeStruct((N, H), jnp.float32),
            jax.ShapeDtypeStruct((N, 8), jnp.float32),
        ],
    )(h, coord, g, b, w1r, w1c)


def _k2_body(pr, pc, c8, row, col, gr, gc, cr, cc,
             rowb, colb, grb, gcb, crb, ccb, sem):
    wid = lax.axis_index("s") * _NC + lax.axis_index("c")
    base = wid * _EPW
    pltpu.sync_copy(row.at[pl.ds(base, _EPW)], rowb)
    pltpu.sync_copy(col.at[pl.ds(base, _EPW)], colb)

    def chunk(i, carry):
        off = i * _CG
        idx_r = rowb.at[pl.ds(off, _CG)]
        idx_c = colb.at[pl.ds(off, _CG)]
        c1 = pltpu.async_copy(pr.at[idx_r], grb, sem)
        c2 = pltpu.async_copy(pc.at[idx_c], gcb, sem)
        c3 = pltpu.async_copy(c8.at[idx_r], crb, sem)
        c4 = pltpu.async_copy(c8.at[idx_c], ccb, sem)
        c1.wait()
        c2.wait()
        c3.wait()
        c4.wait()
        pltpu.sync_copy(grb, gr.at[pl.ds(base + off, _CG)])
        pltpu.sync_copy(gcb, gc.at[pl.ds(base + off, _CG)])
        pltpu.sync_copy(crb, cr.at[pl.ds(base + off, _CG)])
        pltpu.sync_copy(ccb, cc.at[pl.ds(base + off, _CG)])
        return carry

    lax.fori_loop(0, _NCH, chunk, 0)


def _k2(pr, pc, c8, row, col):
    mesh = plsc.VectorSubcoreMesh(core_axis_name="c", subcore_axis_name="s")
    f = pl.kernel(
        _k2_body,
        out_type=[
            jax.ShapeDtypeStruct((E, H), jnp.float32),
            jax.ShapeDtypeStruct((E, H), jnp.float32),
            jax.ShapeDtypeStruct((E, 8), jnp.float32),
            jax.ShapeDtypeStruct((E, 8), jnp.float32),
        ],
        mesh=mesh,
        scratch_types=[
            pltpu.VMEM((_EPW,), jnp.int32),
            pltpu.VMEM((_EPW,), jnp.int32),
            pltpu.VMEM((_CG, H), jnp.float32),
            pltpu.VMEM((_CG, H), jnp.float32),
            pltpu.VMEM((_CG, 8), jnp.float32),
            pltpu.VMEM((_CG, 8), jnp.float32),
            pltpu.SemaphoreType.DMA,
        ],
        compiler_params=pltpu.CompilerParams(use_tc_tiling_on_sc=False),
    )
    return f(pr, pc, c8, row, col)


NPAD = 10240
_TSL = NPAD // _NS
_CS = 80


def _k4_body(ef, tr8, row, z128, z8, aggp, trp,
             idxb, efb, trb, agg_sp, tr_sp):
    c = lax.axis_index("c")
    s = lax.axis_index("s")
    wid = s * _NC + c
    base = wid * _EPW

    pltpu.sync_copy(z128, agg_sp.at[pl.ds(s * _TSL, _TSL)])
    pltpu.sync_copy(z8, tr_sp.at[pl.ds(s * _TSL, _TSL)])
    plsc.subcore_barrier()

    def chunk(i, carry):
        off = base + i * _CS
        pltpu.sync_copy(row.at[pl.ds(off, _CS)], idxb)
        pltpu.sync_copy(ef.at[pl.ds(off, _CS)], efb)
        pltpu.sync_copy(tr8.at[pl.ds(off, _CS)], trb)
        pltpu.sync_copy(efb, agg_sp.at[idxb], add=True)
        pltpu.sync_copy(trb, tr_sp.at[idxb], add=True)
        return carry

    lax.fori_loop(0, _EPW // _CS, chunk, 0)
    plsc.subcore_barrier()

    pltpu.sync_copy(agg_sp.at[pl.ds(s * _TSL, _TSL)],
                    aggp.at[c, pl.ds(s * _TSL, _TSL)])
    pltpu.sync_copy(tr_sp.at[pl.ds(s * _TSL, _TSL)],
                    trp.at[c, pl.ds(s * _TSL, _TSL)])


def _k4(ef, tr8, row, z128, z8):
    mesh = plsc.VectorSubcoreMesh(core_axis_name="c", subcore_axis_name="s")
    f = pl.kernel(
        _k4_body,
        out_type=[
            jax.ShapeDtypeStruct((2, NPAD, H), jnp.float32),
            jax.ShapeDtypeStruct((2, NPAD, 8), jnp.float32),
        ],
        mesh=mesh,
        scratch_types=[
            pltpu.VMEM((_CS,), jnp.int32),
            pltpu.VMEM((_CS, H), jnp.float32),
            pltpu.VMEM((_CS, 8), jnp.float32),
            pltpu.VMEM_SHARED((NPAD, H), jnp.float32),
            pltpu.VMEM_SHARED((NPAD, 8), jnp.float32),
        ],
        compiler_params=pltpu.CompilerParams(use_tc_tiling_on_sc=False),
    )
    return f(ef, tr8, row, z128, z8)


def _k3_body(gr_ref, gc_ref, cr_ref, cc_ref, ea_ref,
             wrad_ref, w1e_ref, b1_ref, w2_ref, b2_ref,
             lng_ref, lnb_ref, cw1_ref, cb1_ref, cw2_ref,
             ef_ref, tr_ref):
    cdiff = cr_ref[:] - cc_ref[:]
    radial = jnp.sum(cdiff * cdiff, axis=1, keepdims=True)
    pre1 = (gr_ref[:] + gc_ref[:]
            + radial * wrad_ref[:]
            + jnp.dot(ea_ref[:], w1e_ref[:], preferred_element_type=jnp.float32)
            + b1_ref[:])
    x1 = _silu(pre1)
    x2 = _silu(jnp.dot(x1, w2_ref[:], preferred_element_type=jnp.float32) + b2_ref[:])
    ef = _ln(x2, lng_ref[:], lnb_ref[:])
    ef_ref[:] = ef
    cm = jnp.dot(_silu(jnp.dot(ef, cw1_ref[:], preferred_element_type=jnp.float32) + cb1_ref[:]),
                 cw2_ref[:], preferred_element_type=jnp.float32)
    tr = cdiff * cm
    col = jax.lax.broadcasted_iota(jnp.int32, (BE, 8), 1)
    tr_ref[:] = jnp.where(col == 3, 1.0, tr)


def _k3(gr, gc, cr, cc, ea, wrad, w1e, b1, w2, b2, lng, lnb, cw1, cb1, cw2):
    grid = E // BE
    full = lambda shape: pl.BlockSpec(shape, lambda i: tuple(0 for _ in shape))
    return pl.pallas_call(
        _k3_body,
        grid=(grid,),
        in_specs=[
            pl.BlockSpec((BE, H), lambda i: (i, 0)),
            pl.BlockSpec((BE, H), lambda i: (i, 0)),
            pl.BlockSpec((BE, 8), lambda i: (i, 0)),
            pl.BlockSpec((BE, 8), lambda i: (i, 0)),
            pl.BlockSpec((BE, DE), lambda i: (i, 0)),
            full((1, H)),
            full((DE, H)),
            full((H,)),
            full((H, H)),
            full((H,)),
            full((H,)),
            full((H,)),
            full((H, H)),
            full((H,)),
            full((H, 1)),
        ],
        out_specs=[
            pl.BlockSpec((BE, H), lambda i: (i, 0)),
            pl.BlockSpec((BE, 8), lambda i: (i, 0)),
        ],
        out_shape=[
            jax.ShapeDtypeStruct((E, H), jnp.float32),
            jax.ShapeDtypeStruct((E, 8), jnp.float32),
        ],
    )(gr, gc, cr, cc, ea, wrad, w1e, b1, w2, b2, lng, lnb, cw1, cb1, cw2)


def _k5_body(h_ref, hn_ref, agg_ref, tr_ref, coord_ref,
             w1h_ref, w1a_ref, b1_ref, w2_ref, b2_ref,
             hout_ref, cout_ref):
    agg = agg_ref[0] + agg_ref[1]
    tr = tr_ref[0] + tr_ref[1]
    pre = (jnp.dot(hn_ref[:], w1h_ref[:], preferred_element_type=jnp.float32)
           + jnp.dot(agg, w1a_ref[:], preferred_element_type=jnp.float32)
           + b1_ref[:])
    nh = jnp.dot(_silu(pre), w2_ref[:], preferred_element_type=jnp.float32) + b2_ref[:]
    hout_ref[:] = h_ref[:] + nh
    cnt = jnp.maximum(tr[:, 3:4], 1.0)
    cout_ref[:] = coord_ref[:] + tr[:, 0:3] / cnt


def _k5(h, hn, aggp, trp, coord, w1h, w1a, b1, w2, b2):
    grid = N // BN
    full = lambda shape: pl.BlockSpec(shape, lambda i: tuple(0 for _ in shape))
    return pl.pallas_call(
        _k5_body,
        grid=(grid,),
        in_specs=[
            pl.BlockSpec((BN, D), lambda i: (i, 0)),
            pl.BlockSpec((BN, D), lambda i: (i, 0)),
            pl.BlockSpec((2, BN, H), lambda i: (0, i, 0)),
            pl.BlockSpec((2, BN, 8), lambda i: (0, i, 0)),
            pl.BlockSpec((BN, 3), lambda i: (i, 0)),
            full((D, H)),
            full((H, H)),
            full((H,)),
            full((H, D)),
            full((D,)),
        ],
        out_specs=[
            pl.BlockSpec((BN, D), lambda i: (i, 0)),
            pl.BlockSpec((BN, 3), lambda i: (i, 0)),
        ],
        out_shape=[
            jax.ShapeDtypeStruct((N, D), jnp.float32),
            jax.ShapeDtypeStruct((N, 3), jnp.float32),
        ],
    )(h, hn, aggp, trp, coord, w1h, w1a, b1, w2, b2)


def kernel(h, edge_index, coord, edge_attr, node_ln_g, node_ln_b,
           edge_ln_g, edge_ln_b, e_W1, e_b1, e_W2, e_b2,
           n_W1, n_b1, n_W2, n_b2, c_W1, c_b1, c_W2):
    row = edge_index[0]
    col = edge_index[1]
    w1r = e_W1[0:D]
    w1c = e_W1[D:2 * D]
    wrad = e_W1[2 * D:2 * D + 1]
    w1e = e_W1[2 * D + 1:]

    hn, pr, pc, c8 = _k1(h, coord, node_ln_g, node_ln_b, w1r, w1c)

    gr, gc, cr, cc = _k2(pr, pc, c8, row, col)

    ef, tr8 = _k3(gr, gc, cr, cc, edge_attr, wrad, w1e, e_b1, e_W2, e_b2,
                  edge_ln_g, edge_ln_b, c_W1, c_b1, c_W2)

    z128 = jnp.zeros((_TSL, H), jnp.float32)
    z8 = jnp.zeros((_TSL, 8), jnp.float32)
    aggp, trp = _k4(ef, tr8, row, z128, z8)

    h_out, coord_out = _k5(h, hn, aggp, trp, coord,
                           n_W1[0:D], n_W1[D:], n_b1, n_W2, n_b2)
    return (h_out, coord_out, edge_attr)


# speedup vs baseline: 4.8626x; 1.2958x over previous
import functools

import jax
import jax.numpy as jnp
from jax import lax
from jax.experimental import pallas as pl
from jax.experimental.pallas import tpu as pltpu
from jax.experimental.pallas import tpu_sc as plsc

N = 10000
E = 320000
D = 128
H = 128
DE = 16

BN = 2000
BE = 2560

_NC = 2
_NS = 16
_NW = _NC * _NS
_CG = 80
_EPW = E // _NW
_NCH = _EPW // _CG
_L = 16


def _silu(x):
    return x / (1.0 + jnp.exp(-x))


def _ln(x, g, b, eps=1e-5):
    mu = jnp.mean(x, axis=-1, keepdims=True)
    var = jnp.mean((x - mu) ** 2, axis=-1, keepdims=True)
    return (x - mu) * jax.lax.rsqrt(var + eps) * g + b


def _k1_body(h_ref, g_ref, b_ref, w1r_ref, w1c_ref,
             hn_ref, pr_ref, pc_ref):
    h = h_ref[:]
    hn = _ln(h, g_ref[:], b_ref[:])
    hn_ref[:] = hn
    pr_ref[:] = jnp.dot(hn, w1r_ref[:], preferred_element_type=jnp.float32)
    pc_ref[:] = jnp.dot(hn, w1c_ref[:], preferred_element_type=jnp.float32)


def _k1(h, g, b, w1r, w1c):
    grid = N // BN
    return pl.pallas_call(
        _k1_body,
        grid=(grid,),
        in_specs=[
            pl.BlockSpec((BN, D), lambda i: (i, 0)),
            pl.BlockSpec((D,), lambda i: (0,)),
            pl.BlockSpec((D,), lambda i: (0,)),
            pl.BlockSpec((D, H), lambda i: (0, 0)),
            pl.BlockSpec((D, H), lambda i: (0, 0)),
        ],
        out_specs=[
            pl.BlockSpec((BN, D), lambda i: (i, 0)),
            pl.BlockSpec((BN, H), lambda i: (i, 0)),
            pl.BlockSpec((BN, H), lambda i: (i, 0)),
        ],
        out_shape=[
            jax.ShapeDtypeStruct((N, D), jnp.float32),
            jax.ShapeDtypeStruct((N, H), jnp.float32),
            jax.ShapeDtypeStruct((N, H), jnp.float32),
        ],
    )(h, g, b, w1r, w1c)


def _k2_body(pr, pc, cx, cy, cz, row, col, z1, gr, gc, cdt, cntp,
             idxr, idxc, grb, gcb, crx, cry, crz, ccx, ccy, ccz,
             planes, ones, cnt_sp, sem):
    c = lax.axis_index("c")
    s = lax.axis_index("s")
    wid = s * _NC + c
    base = wid * _EPW
    pltpu.sync_copy(z1, cnt_sp.at[pl.ds(s * _TSL, _TSL)])
    for j in range(_CG // _L):
        ones[pl.ds(j * _L, _L)] = jnp.full((_L,), 1.0, jnp.float32)
    plsc.subcore_barrier()

    def chunk(i, carry):
        off = i * _CG
        pltpu.sync_copy(row.at[pl.ds(base + off, _CG)], idxr)
        pltpu.sync_copy(col.at[pl.ds(base + off, _CG)], idxc)
        cps = [
            pltpu.async_copy(pr.at[idxr], grb, sem),
            pltpu.async_copy(pc.at[idxc], gcb, sem),
            pltpu.async_copy(cx.at[idxr], crx, sem),
            pltpu.async_copy(cy.at[idxr], cry, sem),
            pltpu.async_copy(cz.at[idxr], crz, sem),
            pltpu.async_copy(cx.at[idxc], ccx, sem),
            pltpu.async_copy(cy.at[idxc], ccy, sem),
            pltpu.async_copy(cz.at[idxc], ccz, sem),
        ]
        for cp in cps:
            cp.wait()
        for j in range(_CG // _L):
            sj = pl.ds(j * _L, _L)
            so = pl.ds(off + j * _L, _L)
            dx = crx[sj] - ccx[sj]
            dy = cry[sj] - ccy[sj]
            dz = crz[sj] - ccz[sj]
            planes[0, so] = dx
            planes[1, so] = dy
            planes[2, so] = dz
            planes[3, so] = dx * dx + dy * dy + dz * dz
        pltpu.sync_copy(grb, gr.at[pl.ds(base + off, _CG)])
        pltpu.sync_copy(gcb, gc.at[pl.ds(base + off, _CG)])
        pltpu.sync_copy(ones, cnt_sp.at[idxr], add=True)
        return carry

    lax.fori_loop(0, _NCH, chunk, 0)
    for d in range(4):
        pltpu.sync_copy(planes.at[d], cdt.at[d, pl.ds(base, _EPW)])
    plsc.subcore_barrier()
    pltpu.sync_copy(cnt_sp.at[pl.ds(s * _TSL, _TSL)],
                    cntp.at[c, pl.ds(s * _TSL, _TSL)])


def _k2(pr, pc, cx, cy, cz, row, col, z1):
    mesh = plsc.VectorSubcoreMesh(core_axis_name="c", subcore_axis_name="s")
    f = pl.kernel(
        _k2_body,
        out_type=[
            jax.ShapeDtypeStruct((E, H), jnp.float32),
            jax.ShapeDtypeStruct((E, H), jnp.float32),
            jax.ShapeDtypeStruct((4, E), jnp.float32),
            jax.ShapeDtypeStruct((2, NPAD), jnp.float32),
        ],
        mesh=mesh,
        scratch_types=[
            pltpu.VMEM((_CG,), jnp.int32),
            pltpu.VMEM((_CG,), jnp.int32),
            pltpu.VMEM((_CG, H), jnp.float32),
            pltpu.VMEM((_CG, H), jnp.float32),
            pltpu.VMEM((_CG,), jnp.float32),
            pltpu.VMEM((_CG,), jnp.float32),
            pltpu.VMEM((_CG,), jnp.float32),
            pltpu.VMEM((_CG,), jnp.float32),
            pltpu.VMEM((_CG,), jnp.float32),
            pltpu.VMEM((_CG,), jnp.float32),
            pltpu.VMEM((4, _EPW), jnp.float32),
            pltpu.VMEM((_CG,), jnp.float32),
            pltpu.VMEM_SHARED((NPAD,), jnp.float32),
            pltpu.SemaphoreType.DMA,
        ],
        compiler_params=pltpu.CompilerParams(use_tc_tiling_on_sc=False),
    )
    return f(pr, pc, cx, cy, cz, row, col, z1)


NPAD = 10240
_TSL = NPAD // _NS
NAGG = 10000
_TSA = NAGG // _NS
_CS = 80


def _k4_body(ef, trt, row, z128, z1, aggp, trp,
             idxb, efb, trpl, agg_sp, t0_sp, t1_sp, t2_sp):
    c = lax.axis_index("c")
    s = lax.axis_index("s")
    wid = s * _NC + c
    base = wid * _EPW
    tsp = [t0_sp, t1_sp, t2_sp]

    pltpu.sync_copy(z128, agg_sp.at[pl.ds(s * _TSA, _TSA)])
    for d in range(3):
        pltpu.sync_copy(z1, tsp[d].at[pl.ds(s * _TSL, _TSL)])
        pltpu.sync_copy(trt.at[d, pl.ds(base, _EPW)], trpl.at[d])
    plsc.subcore_barrier()

    def chunk(i, carry):
        off = base + i * _CS
        pltpu.sync_copy(row.at[pl.ds(off, _CS)], idxb)
        pltpu.sync_copy(ef.at[pl.ds(off, _CS)], efb)
        pltpu.sync_copy(efb, agg_sp.at[idxb], add=True)
        for d in range(3):
            pltpu.sync_copy(trpl.at[d, pl.ds(i * _CS, _CS)],
                            tsp[d].at[idxb], add=True)
        return carry

    lax.fori_loop(0, _EPW // _CS, chunk, 0)
    plsc.subcore_barrier()

    pltpu.sync_copy(agg_sp.at[pl.ds(s * _TSA, _TSA)],
                    aggp.at[c, pl.ds(s * _TSA, _TSA)])
    for d in range(3):
        pltpu.sync_copy(tsp[d].at[pl.ds(s * _TSL, _TSL)],
                        trp.at[c, d, pl.ds(s * _TSL, _TSL)])


def _k4(ef, trt, row, z128, z1):
    mesh = plsc.VectorSubcoreMesh(core_axis_name="c", subcore_axis_name="s")
    f = pl.kernel(
        _k4_body,
        out_type=[
            jax.ShapeDtypeStruct((2, NAGG, H), jnp.float32),
            jax.ShapeDtypeStruct((2, 3, NPAD), jnp.float32),
        ],
        mesh=mesh,
        scratch_types=[
            pltpu.VMEM((_CS,), jnp.int32),
            pltpu.VMEM((_CS, H), jnp.float32),
            pltpu.VMEM((3, _EPW), jnp.float32),
            pltpu.VMEM_SHARED((NAGG, H), jnp.float32),
            pltpu.VMEM_SHARED((NPAD,), jnp.float32),
            pltpu.VMEM_SHARED((NPAD,), jnp.float32),
            pltpu.VMEM_SHARED((NPAD,), jnp.float32),
        ],
        compiler_params=pltpu.CompilerParams(use_tc_tiling_on_sc=False),
    )
    return f(ef, trt, row, z128, z1)


def _k3_body(gr_ref, gc_ref, cd_ref, ea_ref,
             w8_ref, w1e_ref, b1_ref, w2_ref, b2_ref,
             lng_ref, lnb_ref, cw1_ref, cb1_ref, cw2_ref,
             ef_ref, trt_ref):
    cd = cd_ref[:]
    pre1 = (gr_ref[:] + gc_ref[:]
            + jax.lax.dot_general(cd, w8_ref[:], (((0,), (0,)), ((), ())),
                                  preferred_element_type=jnp.float32)
            + jax.lax.dot_general(ea_ref[:], w1e_ref[:], (((0,), (0,)), ((), ())),
                                  preferred_element_type=jnp.float32)
            + b1_ref[:])
    x1 = _silu(pre1)
    x2 = _silu(jnp.dot(x1, w2_ref[:], preferred_element_type=jnp.float32) + b2_ref[:])
    ef = _ln(x2, lng_ref[:], lnb_ref[:])
    ef_ref[:] = ef
    s2 = _silu(jnp.dot(ef, cw1_ref[:], preferred_element_type=jnp.float32) + cb1_ref[:])
    cm_row = jax.lax.dot_general(cw2_ref[:], s2, (((0,), (1,)), ((), ())),
                                 preferred_element_type=jnp.float32)
    trt_ref[:] = cd * cm_row


def _k3(gr, gc, cdt, ea_t, w8, w1e, b1, w2, b2, lng, lnb, cw1, cb1, cw2):
    grid = E // BE
    full = lambda shape: pl.BlockSpec(shape, lambda i: tuple(0 for _ in shape))
    return pl.pallas_call(
        _k3_body,
        grid=(grid,),
        in_specs=[
            pl.BlockSpec((BE, H), lambda i: (i, 0)),
            pl.BlockSpec((BE, H), lambda i: (i, 0)),
            pl.BlockSpec((4, BE), lambda i: (0, i)),
            pl.BlockSpec((DE, BE), lambda i: (0, i)),
            full((4, H)),
            full((DE, H)),
            full((H,)),
            full((H, H)),
            full((H,)),
            full((H,)),
            full((H,)),
            full((H, H)),
            full((H,)),
            full((H, 1)),
        ],
        out_specs=[
            pl.BlockSpec((BE, H), lambda i: (i, 0)),
            pl.BlockSpec((4, BE), lambda i: (0, i)),
        ],
        out_shape=[
            jax.ShapeDtypeStruct((E, H), jnp.float32),
            jax.ShapeDtypeStruct((4, E), jnp.float32),
        ],
    )(gr, gc, cdt, ea_t, w8, w1e, b1, w2, b2, lng, lnb, cw1, cb1, cw2)


def _k5_body(h_ref, hn_ref, agg_ref,
             w1h_ref, w1a_ref, b1_ref, w2_ref, b2_ref,
             hout_ref):
    agg = agg_ref[0] + agg_ref[1]
    pre = (jnp.dot(hn_ref[:], w1h_ref[:], preferred_element_type=jnp.float32)
           + jnp.dot(agg, w1a_ref[:], preferred_element_type=jnp.float32)
           + b1_ref[:])
    nh = jnp.dot(_silu(pre), w2_ref[:], preferred_element_type=jnp.float32) + b2_ref[:]
    hout_ref[:] = h_ref[:] + nh


def _k5(h, hn, aggp, w1h, w1a, b1, w2, b2):
    grid = N // BN
    full = lambda shape: pl.BlockSpec(shape, lambda i: tuple(0 for _ in shape))
    return pl.pallas_call(
        _k5_body,
        grid=(grid,),
        in_specs=[
            pl.BlockSpec((BN, D), lambda i: (i, 0)),
            pl.BlockSpec((BN, D), lambda i: (i, 0)),
            pl.BlockSpec((2, BN, H), lambda i: (0, i, 0)),
            full((D, H)),
            full((H, H)),
            full((H,)),
            full((H, D)),
            full((D,)),
        ],
        out_specs=[
            pl.BlockSpec((BN, D), lambda i: (i, 0)),
        ],
        out_shape=[
            jax.ShapeDtypeStruct((N, D), jnp.float32),
        ],
    )(h, hn, aggp, w1h, w1a, b1, w2, b2)


def _k6_body(trp_ref, cnt_ref, ct_ref, cout_ref):
    tr = trp_ref[0] + trp_ref[1]
    cnt = jnp.maximum(cnt_ref[0:1, :N] + cnt_ref[1:2, :N], 1.0)
    cout_ref[:] = ct_ref[:] + tr[0:3, :N] / cnt


def _k6(trp, cntp, coord_t):
    return pl.pallas_call(
        _k6_body,
        out_shape=jax.ShapeDtypeStruct((3, N), jnp.float32),
    )(trp, cntp, coord_t)


def kernel(h, edge_index, coord, edge_attr, node_ln_g, node_ln_b,
           edge_ln_g, edge_ln_b, e_W1, e_b1, e_W2, e_b2,
           n_W1, n_b1, n_W2, n_b2, c_W1, c_b1, c_W2):
    row = edge_index[0]
    col = edge_index[1]
    w1r = e_W1[0:D]
    w1c = e_W1[D:2 * D]
    wrad = e_W1[2 * D]
    w1e = e_W1[2 * D + 1:]
    w8 = jnp.zeros((4, H), jnp.float32).at[3].set(wrad)
    ea_t = edge_attr.T
    coord_t = coord.T

    hn, pr, pc = _k1(h, node_ln_g, node_ln_b, w1r, w1c)

    z1 = jnp.zeros((_TSL,), jnp.float32)
    gr, gc, cdt, cntp = _k2(pr, pc, coord_t[0], coord_t[1], coord_t[2],
                            row, col, z1)

    ef, trt = _k3(gr, gc, cdt, ea_t, w8, w1e, e_b1, e_W2, e_b2,
                  edge_ln_g, edge_ln_b, c_W1, c_b1, c_W2)

    z128 = jnp.zeros((_TSA, H), jnp.float32)
    aggp, trp = _k4(ef, trt, row, z128, z1)

    (h_out,) = _k5(h, hn, aggp, n_W1[0:D], n_W1[D:], n_b1, n_W2, n_b2)
    coord_out_t = _k6(trp, cntp, coord_t)
    return (h_out, coord_out_t.T, edge_attr)
